# Initial kernel scaffold; baseline (speedup 1.0000x reference)
import jax
import jax.numpy as jnp
from jax.experimental import pallas as pl


def kernel(feat_atom, feat_bond, edge_index_a2b, edge_index_b2a, params):
    raise NotImplementedError("write your pallas kernel here")


# speedup vs baseline: 19.4414x; 19.4414x over previous
import functools

import jax
import jax.numpy as jnp
from jax import lax
from jax.experimental import pallas as pl
from jax.experimental.pallas import tpu as pltpu
from jax.experimental.pallas import tpu_sc as plsc

N_NODE = 100000
N_PAD = 100352
E_EDGE = 200000
E_PAD = 204800
NC, NS, NW = 2, 16, 32
EPW = E_PAD // NW
CH = 128
NCHUNK = EPW // CH
RPT = N_PAD // NS
ZROWS = 128
HID = 64
HEADS = 8
FH = 8
NEG = 0.2
RB = 512

_f32 = jnp.float32


def _vperm(x, idx):
    return lax.gather(
        x, idx[:, None],
        dimension_numbers=lax.GatherDimensionNumbers(
            offset_dims=(), collapsed_slice_dims=(0,), start_index_map=(0,)),
        slice_sizes=(1,),
        mode=lax.GatherScatterMode.PROMISE_IN_BOUNDS)


@functools.cache
def _sc_mesh():
    return plsc.VectorSubcoreMesh(
        core_axis_name="c", subcore_axis_name="s",
        num_cores=NC, num_subcores=NS)


def _sc_edge_softmax(t_hbm, t2_hbm, src_hbm, dst_hbm, bv_hbm,
                     ex_hbm, s_hbm,
                     idx_s, idx_d, srow, drow, exb, zbuf, bvr, sacc,
                     sem_a, sem_b):
    cid = lax.axis_index("c")
    sid = lax.axis_index("s")
    wid = sid * NC + cid

    zero16 = jnp.zeros((16,), _f32)
    for r in range(ZROWS):
        zbuf[r] = zero16
    rbase = sid * RPT

    def zfill(i, _):
        pltpu.sync_copy(zbuf, sacc.at[pl.ds(rbase + i * ZROWS, ZROWS)])
        return _
    lax.fori_loop(0, RPT // ZROWS, zfill, 0)

    pltpu.sync_copy(bv_hbm, bvr)
    bv = bvr[...]
    lane = lax.iota(jnp.int32, 16)
    lo8 = lane < 8

    plsc.subcore_barrier()

    def chunk(i, _):
        base = wid * EPW + i * CH
        pltpu.sync_copy(src_hbm.at[pl.ds(base, CH)], idx_s)
        pltpu.sync_copy(dst_hbm.at[pl.ds(base, CH)], idx_d)
        ca = pltpu.async_copy(t_hbm.at[idx_s], srow, sem_a)
        cb = pltpu.async_copy(t2_hbm.at[idx_d], drow, sem_b)
        ca.wait()
        cb.wait()
        for j in range(CH):
            e = srow[j] + drow[j]
            e = jnp.where(e >= 0.0, e, e * NEG)
            ex = jnp.exp(e - bv)
            exb[j] = jnp.where(lo8, ex, 0.0)
        pltpu.sync_copy(exb, ex_hbm.at[pl.ds(base, CH)])
        pltpu.sync_copy(exb, sacc.at[idx_d], add=True)
        return _
    lax.fori_loop(0, NCHUNK, chunk, 0)

    plsc.subcore_barrier()
    pltpu.sync_copy(sacc.at[pl.ds(rbase, RPT)],
                    s_hbm.at[cid, pl.ds(rbase, RPT)])


@functools.cache
def _edge_softmax_kernel():
    return pl.kernel(
        _sc_edge_softmax,
        out_type=[jax.ShapeDtypeStruct((E_PAD, 16), _f32),
                  jax.ShapeDtypeStruct((NC, N_PAD, 16), _f32)],
        mesh=_sc_mesh(),
        scratch_types=[
            pltpu.VMEM((CH,), jnp.int32),
            pltpu.VMEM((CH,), jnp.int32),
            pltpu.VMEM((CH, 16), _f32),
            pltpu.VMEM((CH, 16), _f32),
            pltpu.VMEM((CH, 16), _f32),
            pltpu.VMEM((ZROWS, 16), _f32),
            pltpu.VMEM((16,), _f32),
            pltpu.VMEM_SHARED((N_PAD, 16), _f32),
            pltpu.SemaphoreType.DMA,
            pltpu.SemaphoreType.DMA,
        ],
        compiler_params=pltpu.CompilerParams(use_tc_tiling_on_sc=False),
    )


def _sc_aggregate(ex_hbm, src_hbm, dst_hbm, z0_hbm, z1_hbm, z2_hbm, z3_hbm,
                  o_hbm,
                  idx_s, idx_d, exb, zrow, msg, zbuf, oacc, sem_a):
    cid = lax.axis_index("c")
    sid = lax.axis_index("s")
    wid = sid * NC + cid

    zero16 = jnp.zeros((16,), _f32)
    for r in range(ZROWS):
        zbuf[r] = zero16
    rbase = sid * RPT
    lane = lax.iota(jnp.int32, 16)

    zq_refs = (z0_hbm, z1_hbm, z2_hbm, z3_hbm)
    for q in range(4):
        def zfill(i, _):
            pltpu.sync_copy(zbuf, oacc.at[pl.ds(rbase + i * ZROWS, ZROWS)])
            return _
        lax.fori_loop(0, RPT // ZROWS, zfill, 0)
        plsc.subcore_barrier()

        col_idx = jnp.where(lane < 8, 2 * q, 2 * q + 1)

        def chunk(i, _, q=q, col_idx=col_idx):
            base = wid * EPW + i * CH
            pltpu.sync_copy(src_hbm.at[pl.ds(base, CH)], idx_s)
            pltpu.sync_copy(dst_hbm.at[pl.ds(base, CH)], idx_d)
            pltpu.sync_copy(ex_hbm.at[pl.ds(base, CH)], exb)
            pltpu.async_copy(zq_refs[q].at[idx_s], zrow, sem_a).wait()
            for j in range(CH):
                a = _vperm(exb[j], col_idx)
                msg[j] = a * zrow[j]
            pltpu.sync_copy(msg, oacc.at[idx_d], add=True)
            return _
        lax.fori_loop(0, NCHUNK, chunk, 0)

        plsc.subcore_barrier()
        pltpu.sync_copy(oacc.at[pl.ds(rbase, RPT)],
                        o_hbm.at[cid, pl.ds(rbase, RPT), pl.ds(16 * q, 16)])
        plsc.subcore_barrier()


@functools.cache
def _aggregate_kernel():
    return pl.kernel(
        _sc_aggregate,
        out_type=jax.ShapeDtypeStruct((NC, N_PAD, HID), _f32),
        mesh=_sc_mesh(),
        scratch_types=[
            pltpu.VMEM((CH,), jnp.int32),
            pltpu.VMEM((CH,), jnp.int32),
            pltpu.VMEM((CH, 16), _f32),
            pltpu.VMEM((CH, 16), _f32),
            pltpu.VMEM((CH, 16), _f32),
            pltpu.VMEM((ZROWS, 16), _f32),
            pltpu.VMEM_SHARED((N_PAD, 16), _f32),
            pltpu.SemaphoreType.DMA,
        ],
        compiler_params=pltpu.CompilerParams(use_tc_tiling_on_sc=False),
    )


def _edge_softmax_call(*args):
    return _edge_softmax_kernel()(*args)


def _aggregate_call(*args):
    return _aggregate_kernel()(*args)


def _pre_body(hs_ref, hd_ref, ws_ref, wl_ref, wr_ref,
              z0_ref, z1_ref, z2_ref, z3_ref, t_ref, t2_ref):
    hs = hs_ref[...]
    hd = hd_ref[...]
    z = jnp.dot(hs, ws_ref[...], preferred_element_type=_f32)
    z0_ref[...] = z[:, 0:16]
    z1_ref[...] = z[:, 16:32]
    z2_ref[...] = z[:, 32:48]
    z3_ref[...] = z[:, 48:64]
    el = jnp.dot(hs, wl_ref[...], preferred_element_type=_f32)
    er = jnp.dot(hd, wr_ref[...], preferred_element_type=_f32)
    t_ref[...] = jnp.concatenate([el, er], axis=1)
    t2_ref[...] = jnp.concatenate([er, el], axis=1)


def _pre_call(hs, hd, ws, wl, wr):
    in_dim = hs.shape[1]
    grid = (N_PAD // RB,)
    q_spec = pl.BlockSpec((RB, 16), lambda i: (i, 0))
    t_spec = pl.BlockSpec((RB, 16), lambda i: (i, 0))
    return pl.pallas_call(
        _pre_body,
        grid=grid,
        in_specs=[
            pl.BlockSpec((RB, in_dim), lambda i: (i, 0)),
            pl.BlockSpec((RB, in_dim), lambda i: (i, 0)),
            pl.BlockSpec((in_dim, HID), lambda i: (0, 0)),
            pl.BlockSpec((in_dim, HEADS), lambda i: (0, 0)),
            pl.BlockSpec((in_dim, HEADS), lambda i: (0, 0)),
        ],
        out_specs=[q_spec, q_spec, q_spec, q_spec, t_spec, t_spec],
        out_shape=[jax.ShapeDtypeStruct((N_PAD, 16), _f32)] * 4
        + [jax.ShapeDtypeStruct((N_PAD, 16), _f32)] * 2,
    )(hs, hd, ws, wl, wr)


def _post_body(o0_ref, o1_ref, s0_ref, s1_ref, r_ref, hprev_ref, out_ref):
    raw = o0_ref[...] + o1_ref[...]
    s = s0_ref[...][:, 0:8] + s1_ref[...][:, 0:8]
    sinv = 1.0 / (s + 1e-9)
    scale = jnp.dot(sinv, r_ref[...], preferred_element_type=_f32)
    out = raw * scale
    if hprev_ref is not None:
        out = out + hprev_ref[...]
    out_ref[...] = jnp.where(out > 0.0, out, jnp.exp(out) - 1.0)


def _post_call(o_slabs, s_slabs, rmat, hprev):
    o0, o1 = o_slabs
    s0, s1 = s_slabs
    grid = (N_PAD // RB,)
    body = _post_body if hprev is not None else (
        lambda o0r, o1r, s0r, s1r, rr, outr:
        _post_body(o0r, o1r, s0r, s1r, rr, None, outr))
    in_specs = [
        pl.BlockSpec((RB, HID), lambda i: (i, 0)),
        pl.BlockSpec((RB, HID), lambda i: (i, 0)),
        pl.BlockSpec((RB, 16), lambda i: (i, 0)),
        pl.BlockSpec((RB, 16), lambda i: (i, 0)),
        pl.BlockSpec((HEADS, HID), lambda i: (0, 0)),
    ]
    args = [o0, o1, s0, s1, rmat]
    if hprev is not None:
        in_specs.append(pl.BlockSpec((RB, HID), lambda i: (i, 0)))
        args.append(hprev)
    return pl.pallas_call(
        body,
        grid=grid,
        in_specs=in_specs,
        out_specs=pl.BlockSpec((RB, HID), lambda i: (i, 0)),
        out_shape=jax.ShapeDtypeStruct((N_PAD, HID), _f32),
    )(*args)


def _fc_body(h_ref, w1_ref, b1_ref, w2_ref, b2_ref, w3_ref, out_ref):
    x = h_ref[...]
    x = jnp.dot(x, w1_ref[...], preferred_element_type=_f32) + b1_ref[...]
    x = jnp.maximum(x, 0.0)
    x = jnp.dot(x, w2_ref[...], preferred_element_type=_f32) + b2_ref[...]
    x = jnp.maximum(x, 0.0)
    out_ref[...] = jnp.dot(x, w3_ref[...], preferred_element_type=_f32)


def _fc_call(h, w1, b1, w2, b2, w3p):
    grid = (N_PAD // RB,)
    return pl.pallas_call(
        _fc_body,
        grid=grid,
        in_specs=[
            pl.BlockSpec((RB, HID), lambda i: (i, 0)),
            pl.BlockSpec((HID, HID), lambda i: (0, 0)),
            pl.BlockSpec((1, HID), lambda i: (0, 0)),
            pl.BlockSpec((HID, HID), lambda i: (0, 0)),
            pl.BlockSpec((1, HID), lambda i: (0, 0)),
            pl.BlockSpec((HID, 8), lambda i: (0, 0)),
        ],
        out_specs=pl.BlockSpec((RB, 8), lambda i: (i, 0)),
        out_shape=jax.ShapeDtypeStruct((N_PAD, 8), _f32),
    )(h, w1, b1, w2, b2, w3p)


def kernel(feat_atom, feat_bond, edge_index_a2b, edge_index_b2a, params):
    attn_order = ("atom", "bond")
    src_type = {"atom": "bond", "bond": "atom"}

    pad_n = N_PAD - N_NODE
    h = {
        "atom": jnp.pad(feat_atom, ((0, pad_n), (0, 0))),
        "bond": jnp.pad(feat_bond, ((0, pad_n), (0, 0))),
    }
    pad_e = E_PAD - E_EDGE
    edges = {}
    for nt, ei in (("atom", edge_index_b2a), ("bond", edge_index_a2b)):
        src = jnp.pad(ei[0].astype(jnp.int32), (0, pad_e),
                      constant_values=N_NODE)
        dst = jnp.pad(ei[1].astype(jnp.int32), (0, pad_e),
                      constant_values=N_NODE)
        edges[nt] = (src, dst)

    rmat = jnp.repeat(jnp.eye(HEADS, dtype=_f32), FH, axis=1)

    for l, layer_p in enumerate(params["gat"]):
        slabs = {}
        for nt in attn_order:
            st = src_type[nt]
            p = layer_p[nt]
            a_l = jnp.zeros((HID, HEADS), _f32)
            a_l = a_l.at[jnp.arange(HID), jnp.arange(HID) // FH].set(
                p["attn_l"].reshape(HID))
            a_r = jnp.zeros((HID, HEADS), _f32)
            a_r = a_r.at[jnp.arange(HID), jnp.arange(HID) // FH].set(
                p["attn_r"].reshape(HID))
            wl = jnp.dot(p["W_src"], a_l)
            wr = jnp.dot(p["W_dst"], a_r)

            z0, z1, z2, z3, t, t2 = _pre_call(h[st], h[nt], p["W_src"], wl, wr)

            m = jnp.max(t, axis=0)
            b_sc = jnp.max(m[:8]) + jnp.max(m[8:])
            b_sc = jnp.where(b_sc >= 0.0, b_sc, b_sc * NEG)
            bv = jnp.full((16,), b_sc, _f32)

            src, dst = edges[nt]
            ex, s_out = _edge_softmax_call(t, t2, src, dst, bv)
            o_out = _aggregate_call(ex, src, dst, z0, z1, z2, z3)
            slabs[nt] = (o_out, s_out)

        new_h = {}
        for nt in attn_order:
            o_out, s_out = slabs[nt]
            o0, o1 = o_out[0], o_out[1]
            hprev = h[nt] if l > 0 else None
            new_h[nt] = _post_call((o0, o1), (s_out[0], s_out[1]), rmat, hprev)
        h = new_h

    fc = params["fc"]
    w3p = jnp.pad(fc[2]["W"], ((0, 0), (0, 7)))
    y = _fc_call(h["bond"], fc[0]["W"], fc[0]["b"].reshape(1, HID),
                 fc[1]["W"], fc[1]["b"].reshape(1, HID), w3p)
    return y[:N_NODE, 0:1] + fc[2]["b"].reshape(1, 1)


# speedup vs baseline: 23.3484x; 1.2010x over previous
import functools

import jax
import jax.numpy as jnp
from jax import lax
from jax.experimental import pallas as pl
from jax.experimental.pallas import tpu as pltpu
from jax.experimental.pallas import tpu_sc as plsc

N_NODE = 100000
N_PAD = 100352
E_EDGE = 200000
NC, NS, NW = 2, 16, 32
CH = 128
NCHUNK = 51
E_PAD = NW * NCHUNK * CH
EROWS = E_PAD // CH
RPT = N_PAD // NS
ZROWS = 784
HID = 64
HEADS = 8
FH = 8
NEG = 0.2
RB = 512

_f32 = jnp.float32


def _vperm(x, idx):
    return lax.gather(
        x, idx[:, None],
        dimension_numbers=lax.GatherDimensionNumbers(
            offset_dims=(), collapsed_slice_dims=(0,), start_index_map=(0,)),
        slice_sizes=(1,),
        mode=lax.GatherScatterMode.PROMISE_IN_BOUNDS)


@functools.cache
def _sc_mesh():
    return plsc.VectorSubcoreMesh(
        core_axis_name="c", subcore_axis_name="s",
        num_cores=NC, num_subcores=NS)


def _sc_edge_a(t_hbm, t2_hbm, src_hbm, dst_hbm, bv_hbm,
               ex_hbm, s_hbm,
               idxs2, idxd2, srow, drow, exb, zbuf, bvr,
               sacc, isem, gsem, ssem):
    cid = lax.axis_index("c")
    sid = lax.axis_index("s")
    wid = sid * NC + cid
    rbase = sid * RPT
    lane = lax.iota(jnp.int32, 16)
    lo8 = lane < 8
    zero16 = jnp.zeros((16,), _f32)
    for r in range(ZROWS):
        zbuf[r] = zero16

    pltpu.sync_copy(bv_hbm, bvr)
    bv = bvr[...]

    for k in range(RPT // ZROWS):
        pltpu.async_copy(zbuf, sacc.at[pl.ds(rbase + k * ZROWS, ZROWS)], gsem)
    for k in range(RPT // ZROWS):
        pltpu.make_async_copy(
            zbuf, sacc.at[pl.ds(rbase, ZROWS)], gsem).wait()
    plsc.subcore_barrier()

    def iissue(c, p):
        pltpu.async_copy(src_hbm.at[wid * NCHUNK + c], idxs2.at[p], isem)
        pltpu.async_copy(dst_hbm.at[wid * NCHUNK + c], idxd2.at[p], isem)

    def iwait(p):
        pltpu.make_async_copy(src_hbm.at[0], idxs2.at[p], isem).wait()
        pltpu.make_async_copy(dst_hbm.at[0], idxd2.at[p], isem).wait()

    def gissue(p):
        pltpu.async_copy(t_hbm.at[idxs2.at[p]], srow.at[p], gsem)
        pltpu.async_copy(t2_hbm.at[idxd2.at[p]], drow.at[p], gsem)

    def gwait(p):
        pltpu.make_async_copy(t_hbm.at[idxs2.at[0]], srow.at[p], gsem).wait()
        pltpu.make_async_copy(t2_hbm.at[idxd2.at[0]], drow.at[p], gsem).wait()

    def oissue(c, p):
        base = (wid * NCHUNK + c) * CH
        pltpu.async_copy(exb.at[p], ex_hbm.at[pl.ds(base, CH)], ssem)
        pltpu.sync_copy(exb.at[p], sacc.at[idxd2.at[p]], add=True)

    def odrain(p):
        pltpu.make_async_copy(exb.at[p], ex_hbm.at[pl.ds(0, CH)], ssem).wait()

    def compute(p):
        def grp(k, _):
            for jj in range(16):
                j = k * 16 + jj
                e = srow[p, j] + drow[p, j]
                e = jnp.where(e >= 0.0, e, e * NEG)
                ex = jnp.exp(e - bv)
                exb[p, j] = jnp.where(lo8, ex, 0.0)
            return _
        lax.fori_loop(0, CH // 16, grp, 0)

    iissue(0, 0)
    iissue(1, 1)
    iwait(0)
    gissue(0)

    def loop(g, _):
        for p in (0, 1):
            c = 2 * g + p
            gwait(p)

            @pl.when(g >= 1)
            def _d():
                odrain(p)
            iwait(1 - p)
            gissue(1 - p)
            compute(p)
            oissue(c, p)

            @pl.when(c + 2 <= NCHUNK - 1)
            def _i():
                iissue(c + 2, p)
        return _
    lax.fori_loop(0, (NCHUNK - 1) // 2, loop, 0)
    gwait(0)
    odrain(0)
    compute(0)
    oissue(NCHUNK - 1, 0)
    odrain(1)
    odrain(0)

    plsc.subcore_barrier()
    pltpu.sync_copy(sacc.at[pl.ds(rbase, RPT)],
                    s_hbm.at[cid, pl.ds(rbase, RPT)])


@functools.cache
def _edge_a_kernel():
    return pl.kernel(
        _sc_edge_a,
        out_type=[jax.ShapeDtypeStruct((E_PAD, 16), _f32),
                  jax.ShapeDtypeStruct((NC, N_PAD, 16), _f32)],
        mesh=_sc_mesh(),
        scratch_types=[
            pltpu.VMEM((2, CH), jnp.int32),
            pltpu.VMEM((2, CH), jnp.int32),
            pltpu.VMEM((2, CH, 16), _f32),
            pltpu.VMEM((2, CH, 16), _f32),
            pltpu.VMEM((2, CH, 16), _f32),
            pltpu.VMEM((ZROWS, 16), _f32),
            pltpu.VMEM((16,), _f32),
            pltpu.VMEM_SHARED((N_PAD, 16), _f32),
            pltpu.SemaphoreType.DMA,
            pltpu.SemaphoreType.DMA,
            pltpu.SemaphoreType.DMA,
        ],
        compiler_params=pltpu.CompilerParams(use_tc_tiling_on_sc=False),
    )


def _sc_edge_b(ex_hbm, src_hbm, dst_hbm, z0_hbm, z1_hbm, z2_hbm, z3_hbm,
               o_hbm,
               idxs2, idxd2, exb, zrow, msg, zbuf,
               sacc, isem, gsem, ssem):
    cid = lax.axis_index("c")
    sid = lax.axis_index("s")
    wid = sid * NC + cid
    rbase = sid * RPT
    lane = lax.iota(jnp.int32, 16)
    lo8 = lane < 8
    zero16 = jnp.zeros((16,), _f32)
    for r in range(ZROWS):
        zbuf[r] = zero16

    zq_refs = (z0_hbm, z1_hbm, z2_hbm, z3_hbm)

    def iissue(c, p):
        pltpu.async_copy(src_hbm.at[wid * NCHUNK + c], idxs2.at[p], isem)
        pltpu.async_copy(dst_hbm.at[wid * NCHUNK + c], idxd2.at[p], isem)

    def iwait(p):
        pltpu.make_async_copy(src_hbm.at[0], idxs2.at[p], isem).wait()
        pltpu.make_async_copy(dst_hbm.at[0], idxd2.at[p], isem).wait()

    for q in range(4):
        for k in range(RPT // ZROWS):
            pltpu.async_copy(zbuf, sacc.at[pl.ds(rbase + k * ZROWS, ZROWS)],
                             gsem)
        for k in range(RPT // ZROWS):
            pltpu.make_async_copy(
                zbuf, sacc.at[pl.ds(rbase, ZROWS)], gsem).wait()
        plsc.subcore_barrier()
        col_q = jnp.where(lo8, 2 * q, 2 * q + 1)

        def gissue(c, p, q=q):
            base = (wid * NCHUNK + c) * CH
            pltpu.async_copy(zq_refs[q].at[idxs2.at[p]], zrow.at[p], gsem)
            pltpu.async_copy(ex_hbm.at[pl.ds(base, CH)], exb.at[p], gsem)

        def gwait(p, q=q):
            pltpu.make_async_copy(
                zq_refs[q].at[idxs2.at[0]], zrow.at[p], gsem).wait()
            pltpu.make_async_copy(
                ex_hbm.at[pl.ds(0, CH)], exb.at[p], gsem).wait()

        def missue(p):
            pltpu.sync_copy(msg.at[p], sacc.at[idxd2.at[p]], add=True)

        def compute(p, col_q=col_q):
            def grp(k, _):
                for jj in range(16):
                    j = k * 16 + jj
                    msg[p, j] = _vperm(exb[p, j], col_q) * zrow[p, j]
                return _
            lax.fori_loop(0, CH // 16, grp, 0)

        iissue(0, 0)
        iissue(1, 1)
        iwait(0)
        gissue(0, 0)

        def loop(g, _):
            for p in (0, 1):
                c = 2 * g + p
                gwait(p)
                iwait(1 - p)
                gissue(c + 1, 1 - p)
                compute(p)
                missue(p)

                @pl.when(c + 2 <= NCHUNK - 1)
                def _i():
                    iissue(c + 2, p)
            return _
        lax.fori_loop(0, (NCHUNK - 1) // 2, loop, 0)
        gwait(0)
        compute(0)
        missue(0)

        plsc.subcore_barrier()
        pltpu.sync_copy(sacc.at[pl.ds(rbase, RPT)],
                        o_hbm.at[cid, pl.ds(rbase, RPT), pl.ds(16 * q, 16)])
        plsc.subcore_barrier()


@functools.cache
def _edge_b_kernel():
    return pl.kernel(
        _sc_edge_b,
        out_type=jax.ShapeDtypeStruct((NC, N_PAD, HID), _f32),
        mesh=_sc_mesh(),
        scratch_types=[
            pltpu.VMEM((2, CH), jnp.int32),
            pltpu.VMEM((2, CH), jnp.int32),
            pltpu.VMEM((2, CH, 16), _f32),
            pltpu.VMEM((2, CH, 16), _f32),
            pltpu.VMEM((2, CH, 16), _f32),
            pltpu.VMEM((ZROWS, 16), _f32),
            pltpu.VMEM_SHARED((N_PAD, 16), _f32),
            pltpu.SemaphoreType.DMA,
            pltpu.SemaphoreType.DMA,
            pltpu.SemaphoreType.DMA,
        ],
        compiler_params=pltpu.CompilerParams(use_tc_tiling_on_sc=False),
    )


def _edge_call(t, t2, src, dst, bv, z0, z1, z2, z3):
    ex, s_out = _edge_a_kernel()(t, t2, src, dst, bv)
    o_out = _edge_b_kernel()(ex, src, dst, z0, z1, z2, z3)
    return s_out, o_out


def _pre_body(hs_ref, hd_ref, ws_ref, wl_ref, wr_ref,
              z0_ref, z1_ref, z2_ref, z3_ref, t_ref, t2_ref):
    hs = hs_ref[...]
    hd = hd_ref[...]
    z = jnp.dot(hs, ws_ref[...], preferred_element_type=_f32)
    z0_ref[...] = z[:, 0:16]
    z1_ref[...] = z[:, 16:32]
    z2_ref[...] = z[:, 32:48]
    z3_ref[...] = z[:, 48:64]
    el = jnp.dot(hs, wl_ref[...], preferred_element_type=_f32)
    er = jnp.dot(hd, wr_ref[...], preferred_element_type=_f32)
    t_ref[...] = jnp.concatenate([el, er], axis=1)
    t2_ref[...] = jnp.concatenate([er, el], axis=1)


def _pre_call(hs, hd, ws, wl, wr):
    in_dim = hs.shape[1]
    grid = (N_PAD // RB,)
    q_spec = pl.BlockSpec((RB, 16), lambda i: (i, 0))
    t_spec = pl.BlockSpec((RB, 16), lambda i: (i, 0))
    return pl.pallas_call(
        _pre_body,
        grid=grid,
        in_specs=[
            pl.BlockSpec((RB, in_dim), lambda i: (i, 0)),
            pl.BlockSpec((RB, in_dim), lambda i: (i, 0)),
            pl.BlockSpec((in_dim, HID), lambda i: (0, 0)),
            pl.BlockSpec((in_dim, HEADS), lambda i: (0, 0)),
            pl.BlockSpec((in_dim, HEADS), lambda i: (0, 0)),
        ],
        out_specs=[q_spec, q_spec, q_spec, q_spec, t_spec, t_spec],
        out_shape=[jax.ShapeDtypeStruct((N_PAD, 16), _f32)] * 4
        + [jax.ShapeDtypeStruct((N_PAD, 16), _f32)] * 2,
    )(hs, hd, ws, wl, wr)


def _post_body(o0_ref, o1_ref, s0_ref, s1_ref, r_ref, hprev_ref, out_ref):
    raw = o0_ref[...] + o1_ref[...]
    s = s0_ref[...][:, 0:8] + s1_ref[...][:, 0:8]
    sinv = 1.0 / (s + 1e-9)
    scale = jnp.dot(sinv, r_ref[...], preferred_element_type=_f32)
    out = raw * scale
    if hprev_ref is not None:
        out = out + hprev_ref[...]
    out_ref[...] = jnp.where(out > 0.0, out, jnp.exp(out) - 1.0)


def _post_call(o_slabs, s_slabs, rmat, hprev):
    o0, o1 = o_slabs
    s0, s1 = s_slabs
    grid = (N_PAD // RB,)
    body = _post_body if hprev is not None else (
        lambda o0r, o1r, s0r, s1r, rr, outr:
        _post_body(o0r, o1r, s0r, s1r, rr, None, outr))
    in_specs = [
        pl.BlockSpec((RB, HID), lambda i: (i, 0)),
        pl.BlockSpec((RB, HID), lambda i: (i, 0)),
        pl.BlockSpec((RB, 16), lambda i: (i, 0)),
        pl.BlockSpec((RB, 16), lambda i: (i, 0)),
        pl.BlockSpec((HEADS, HID), lambda i: (0, 0)),
    ]
    args = [o0, o1, s0, s1, rmat]
    if hprev is not None:
        in_specs.append(pl.BlockSpec((RB, HID), lambda i: (i, 0)))
        args.append(hprev)
    return pl.pallas_call(
        body,
        grid=grid,
        in_specs=in_specs,
        out_specs=pl.BlockSpec((RB, HID), lambda i: (i, 0)),
        out_shape=jax.ShapeDtypeStruct((N_PAD, HID), _f32),
    )(*args)


def _fc_body(h_ref, w1_ref, b1_ref, w2_ref, b2_ref, w3_ref, out_ref):
    x = h_ref[...]
    x = jnp.dot(x, w1_ref[...], preferred_element_type=_f32) + b1_ref[...]
    x = jnp.maximum(x, 0.0)
    x = jnp.dot(x, w2_ref[...], preferred_element_type=_f32) + b2_ref[...]
    x = jnp.maximum(x, 0.0)
    out_ref[...] = jnp.dot(x, w3_ref[...], preferred_element_type=_f32)


def _fc_call(h, w1, b1, w2, b2, w3p):
    grid = (N_PAD // RB,)
    return pl.pallas_call(
        _fc_body,
        grid=grid,
        in_specs=[
            pl.BlockSpec((RB, HID), lambda i: (i, 0)),
            pl.BlockSpec((HID, HID), lambda i: (0, 0)),
            pl.BlockSpec((1, HID), lambda i: (0, 0)),
            pl.BlockSpec((HID, HID), lambda i: (0, 0)),
            pl.BlockSpec((1, HID), lambda i: (0, 0)),
            pl.BlockSpec((HID, 8), lambda i: (0, 0)),
        ],
        out_specs=pl.BlockSpec((RB, 8), lambda i: (i, 0)),
        out_shape=jax.ShapeDtypeStruct((N_PAD, 8), _f32),
    )(h, w1, b1, w2, b2, w3p)


def kernel(feat_atom, feat_bond, edge_index_a2b, edge_index_b2a, params):
    attn_order = ("atom", "bond")
    src_type = {"atom": "bond", "bond": "atom"}

    pad_n = N_PAD - N_NODE
    h = {
        "atom": jnp.pad(feat_atom, ((0, pad_n), (0, 0))),
        "bond": jnp.pad(feat_bond, ((0, pad_n), (0, 0))),
    }
    pad_e = E_PAD - E_EDGE
    edges = {}
    for nt, ei in (("atom", edge_index_b2a), ("bond", edge_index_a2b)):
        src = jnp.pad(ei[0].astype(jnp.int32), (0, pad_e),
                      constant_values=N_NODE).reshape(EROWS, CH)
        dst = jnp.pad(ei[1].astype(jnp.int32), (0, pad_e),
                      constant_values=N_NODE).reshape(EROWS, CH)
        edges[nt] = (src, dst)

    rmat = jnp.repeat(jnp.eye(HEADS, dtype=_f32), FH, axis=1)

    for l, layer_p in enumerate(params["gat"]):
        slabs = {}
        for nt in attn_order:
            st = src_type[nt]
            p = layer_p[nt]
            a_l = jnp.zeros((HID, HEADS), _f32)
            a_l = a_l.at[jnp.arange(HID), jnp.arange(HID) // FH].set(
                p["attn_l"].reshape(HID))
            a_r = jnp.zeros((HID, HEADS), _f32)
            a_r = a_r.at[jnp.arange(HID), jnp.arange(HID) // FH].set(
                p["attn_r"].reshape(HID))
            wl = jnp.dot(p["W_src"], a_l)
            wr = jnp.dot(p["W_dst"], a_r)

            z0, z1, z2, z3, t, t2 = _pre_call(h[st], h[nt], p["W_src"], wl, wr)

            m = jnp.max(t, axis=0)
            b_sc = jnp.max(m[:8]) + jnp.max(m[8:])
            b_sc = jnp.where(b_sc >= 0.0, b_sc, b_sc * NEG)
            bv = jnp.full((16,), b_sc, _f32)

            src, dst = edges[nt]
            s_out, o_out = _edge_call(t, t2, src, dst, bv, z0, z1, z2, z3)
            slabs[nt] = (o_out, s_out)

        new_h = {}
        for nt in attn_order:
            o_out, s_out = slabs[nt]
            o0, o1 = o_out[0], o_out[1]
            hprev = h[nt] if l > 0 else None
            new_h[nt] = _post_call((o0, o1), (s_out[0], s_out[1]), rmat, hprev)
        h = new_h

    fc = params["fc"]
    w3p = jnp.pad(fc[2]["W"], ((0, 0), (0, 7)))
    y = _fc_call(h["bond"], fc[0]["W"], fc[0]["b"].reshape(1, HID),
                 fc[1]["W"], fc[1]["b"].reshape(1, HID), w3p)
    return y[:N_NODE, 0:1] + fc[2]["b"].reshape(1, 1)
